# baseline (speedup 1.0000x reference)
import functools

import jax
import jax.numpy as jnp
from jax import lax
from jax.experimental import pallas as pl
from jax.experimental.pallas import tpu as pltpu
from jax.experimental.pallas import tpu_sc as plsc

TOPK = 20
EPS = 1e-5
IDX_CHUNK = 128


def _sc_gather_build(rows_total, rpw, nch, n, nc, v_rows):
    mesh = plsc.VectorSubcoreMesh(core_axis_name="c", subcore_axis_name="s")

    @functools.partial(
        pl.kernel,
        mesh=mesh,
        out_type=jax.ShapeDtypeStruct((rows_total, n), jnp.float32),
        scratch_types=[
            pltpu.VMEM((nch, IDX_CHUNK), jnp.int32),
            pltpu.VMEM((rpw, n), jnp.float32),
            pltpu.SemaphoreType.DMA,
        ],
    )
    def sc_gather(bidx_hbm, didx_hbm, table_hbm, out_hbm, idx_v, rows_v, sem):
        wid = lax.axis_index("s") * nc + lax.axis_index("c")
        graph = wid // 2
        off = graph * v_rows

        @pl.when(wid % 2 == 0)
        def _births():
            pltpu.sync_copy(bidx_hbm.at[graph], idx_v)

        @pl.when(wid % 2 == 1)
        def _deaths():
            pltpu.sync_copy(didx_hbm.at[graph], idx_v)

        for j in range(nch):
            for k in range(IDX_CHUNK // 16):
                sl = (j, pl.ds(k * 16, 16))
                idx_v[sl] = idx_v[sl] + off
        copies = [
            pltpu.async_copy(
                table_hbm.at[idx_v.at[j]],
                rows_v.at[pl.ds(j * IDX_CHUNK, IDX_CHUNK)],
                sem,
            )
            for j in range(nch)
        ]
        for cp in copies:
            cp.wait()
        pltpu.sync_copy(rows_v, out_hbm.at[pl.ds(wid * rpw, rpw)])

    return sc_gather


def _tc_body(ev_ref, w1_ref, b1_ref, w2_ref, b2_ref, w3_ref, b3_ref,
             pw_ref, pb_ref, r_ref, out_ref, v_ref):
    step = pl.program_id(0)
    num_steps = pl.num_programs(0)
    gb = r_ref.shape[0]
    p_pairs = r_ref.shape[1] // 6

    for gi in range(gb):
        ev = ev_ref[gi]
        h1 = jnp.maximum(w1_ref[...] * ev + b1_ref[...], 0.0)
        h2 = lax.dot_general(w2_ref[...], h1, (((1,), (0,)), ((), ())),
                             preferred_element_type=jnp.float32)
        h2 = jnp.maximum(h2 + b2_ref[...], 0.0)
        g0 = lax.dot_general(w3_ref[...], h2, (((1,), (0,)), ((), ())),
                             preferred_element_type=jnp.float32)
        g0 = g0 + b3_ref[...]
        mu = jnp.mean(g0, axis=1, keepdims=True)
        var = jnp.mean((g0 - mu) ** 2, axis=1, keepdims=True)
        g = (g0 - mu) / jnp.sqrt(var + EPS)
        v = lax.dot_general(g, r_ref[gi], (((1,), (1,)), ((), ())),
                            preferred_element_type=jnp.float32)
        v_ref[pl.ds(step * gb + gi, 1), :] = v

    @pl.when(step == num_steps - 1)
    def _finish():
        vall = v_ref[:, :]
        bv = jnp.concatenate(
            [vall[:, c * p_pairs:(c + 1) * p_pairs] for c in range(3)], axis=0)
        dv = jnp.concatenate(
            [vall[:, (3 + c) * p_pairs:(4 + c) * p_pairs] for c in range(3)],
            axis=0)
        rows = bv.shape[0]
        p = jnp.abs(dv - bv)
        iota = lax.broadcasted_iota(jnp.int32, (rows, p_pairs), 1)

        pr = p
        for _ in range(TOPK):
            mx = jnp.max(pr, axis=1, keepdims=True)
            cand = jnp.where(pr == mx, iota, p_pairs)
            fi = jnp.min(cand, axis=1, keepdims=True)
            pr = jnp.where(iota == fi, -1.0, pr)
        msk = (pr < 0.0) & (p >= 0.0)
        mf = msk.astype(jnp.float32)
        pm = p * mf
        lg = jnp.log(1.0 + pm) * mf
        f0 = jnp.sum(pm, axis=1, keepdims=True)
        f1 = jnp.sum(bv * pm, axis=1, keepdims=True)
        f2 = jnp.sum(dv * pm, axis=1, keepdims=True)
        f3 = jnp.sum(bv * lg, axis=1, keepdims=True)
        f4 = jnp.sum(dv * lg, axis=1, keepdims=True)
        f5s = jnp.sum(jnp.exp(pm - 1.0) * (pm > 0).astype(jnp.float32) * mf,
                      axis=1, keepdims=True)
        f5 = jnp.log(jnp.exp(jnp.float32(-1.0)) + f5s) + 1.0
        feats = jnp.concatenate([f0, f1, f2, f3, f4, f5], axis=1)
        f3d = feats.reshape(3, num_steps * gb, 6)
        mu2 = jnp.mean(f3d, axis=1, keepdims=True)
        var2 = jnp.mean((f3d - mu2) ** 2, axis=1, keepdims=True)
        fn = (f3d - mu2) / jnp.sqrt(var2 + EPS)
        t = fn * pw_ref[...][:, None, :]
        s = jnp.sum(jnp.sum(t, axis=2, keepdims=True), axis=0)
        out_ref[:, :] = s + pb_ref[...]


def kernel(eigenvalues, eigenvectors_sq, W1, b1, W2, b2, W3, b3,
           proj_W, proj_b, birth_idx, death_idx):
    L, V, N = eigenvectors_sq.shape
    P = birth_idx.shape[-1]
    planes = W1.shape[1]
    groups = W2.shape[1]
    rows_total = L * 6 * P

    info = plsc.get_sparse_core_info()
    nc = info.num_cores
    nw = nc * info.num_subcores
    rpw = rows_total // nw
    nch = rpw // IDX_CHUNK

    bidx = birth_idx.reshape(L, nch, IDX_CHUNK)
    didx = death_idx.reshape(L, nch, IDX_CHUNK)
    table = eigenvectors_sq.reshape(L * V, N)
    rows = _sc_gather_build(rows_total, rpw, nch, N, nc, V)(bidx, didx, table)
    rows4 = rows.reshape(L, 6 * P, N)

    ev2 = eigenvalues[:, :, 0].reshape(L, 1, N)
    w1t = W1.T
    b1c = b1.reshape(planes, 1)
    w2t = W2.T
    b2c = b2.reshape(groups, 1)
    w3t = W3.T
    b3c = b3.reshape(1, 1)
    pw3 = proj_W.reshape(3, 6)
    pb2 = proj_b.reshape(1, 1)

    gb = 4
    return pl.pallas_call(
        _tc_body,
        grid=(L // gb,),
        in_specs=[
            pl.BlockSpec((gb, 1, N), lambda l: (l, 0, 0)),
            pl.BlockSpec((planes, 1), lambda l: (0, 0)),
            pl.BlockSpec((planes, 1), lambda l: (0, 0)),
            pl.BlockSpec((groups, planes), lambda l: (0, 0)),
            pl.BlockSpec((groups, 1), lambda l: (0, 0)),
            pl.BlockSpec((1, groups), lambda l: (0, 0)),
            pl.BlockSpec((1, 1), lambda l: (0, 0)),
            pl.BlockSpec((3, 6), lambda l: (0, 0)),
            pl.BlockSpec((1, 1), lambda l: (0, 0)),
            pl.BlockSpec((gb, 6 * P, N), lambda l: (l, 0, 0)),
        ],
        out_specs=pl.BlockSpec((L, 1), lambda l: (0, 0)),
        out_shape=jax.ShapeDtypeStruct((L, 1), jnp.float32),
        scratch_shapes=[pltpu.VMEM((L, 6 * P), jnp.float32)],
    )(ev2, w1t, b1c, w2t, b2c, w3t, b3c, pw3, pb2, rows4)


# speedup vs baseline: 1.0067x; 1.0067x over previous
import functools

import jax
import jax.numpy as jnp
from jax import lax
from jax.experimental import pallas as pl
from jax.experimental.pallas import tpu as pltpu
from jax.experimental.pallas import tpu_sc as plsc

TOPK = 20
EPS = 1e-5
IDX_CHUNK = 128


def _sc_gather_build(rows_total, rpw, nch, n, nc):
    mesh = plsc.VectorSubcoreMesh(core_axis_name="c", subcore_axis_name="s")

    @functools.partial(
        pl.kernel,
        mesh=mesh,
        out_type=jax.ShapeDtypeStruct((rows_total, n), jnp.float32),
        scratch_types=[
            pltpu.VMEM((nch, IDX_CHUNK), jnp.int32),
            pltpu.VMEM((rpw, n), jnp.float32),
        ]
        + [pltpu.SemaphoreType.DMA] * (nch + 1),
    )
    def sc_gather(table_hbm, idx_hbm, out_hbm, idx_v, rows_v, *sems):
        gsems, osem = sems[:nch], sems[nch]
        wid = lax.axis_index("s") * nc + lax.axis_index("c")
        base = wid * rpw
        pltpu.sync_copy(idx_hbm.at[wid], idx_v)
        gathers = [
            pltpu.async_copy(
                table_hbm.at[idx_v.at[j]],
                rows_v.at[pl.ds(j * IDX_CHUNK, IDX_CHUNK)],
                gsems[j],
            )
            for j in range(nch)
        ]
        writes = []
        for j in range(nch):
            gathers[j].wait()
            writes.append(
                pltpu.async_copy(
                    rows_v.at[pl.ds(j * IDX_CHUNK, IDX_CHUNK)],
                    out_hbm.at[pl.ds(base + j * IDX_CHUNK, IDX_CHUNK)],
                    osem,
                )
            )
        for w in writes:
            w.wait()

    return sc_gather


def _tc_body(ev_ref, w1_ref, b1_ref, w2_ref, b2_ref, w3_ref, b3_ref,
             pw_ref, pb_ref, r_ref, out_ref, v_ref):
    step = pl.program_id(0)
    num_steps = pl.num_programs(0)
    gb = r_ref.shape[0]
    p_pairs = r_ref.shape[1] // 6

    for gi in range(gb):
        ev = ev_ref[gi]
        h1 = jnp.maximum(w1_ref[...] * ev + b1_ref[...], 0.0)
        h2 = lax.dot_general(w2_ref[...], h1, (((1,), (0,)), ((), ())),
                             preferred_element_type=jnp.float32)
        h2 = jnp.maximum(h2 + b2_ref[...], 0.0)
        g0 = lax.dot_general(w3_ref[...], h2, (((1,), (0,)), ((), ())),
                             preferred_element_type=jnp.float32)
        g0 = g0 + b3_ref[...]
        mu = jnp.mean(g0, axis=1, keepdims=True)
        var = jnp.mean((g0 - mu) ** 2, axis=1, keepdims=True)
        g = (g0 - mu) / jnp.sqrt(var + EPS)
        v = lax.dot_general(g, r_ref[gi], (((1,), (1,)), ((), ())),
                            preferred_element_type=jnp.float32)
        v_ref[pl.ds(step * gb + gi, 1), :] = v

    @pl.when(step == num_steps - 1)
    def _finish():
        vall = v_ref[:, :]
        bv = jnp.concatenate(
            [vall[:, c * p_pairs:(c + 1) * p_pairs] for c in range(3)], axis=0)
        dv = jnp.concatenate(
            [vall[:, (3 + c) * p_pairs:(4 + c) * p_pairs] for c in range(3)],
            axis=0)
        rows = bv.shape[0]
        p = jnp.abs(dv - bv)
        iota = lax.broadcasted_iota(jnp.int32, (rows, p_pairs), 1)

        pr = p
        for _ in range(TOPK):
            mx = jnp.max(pr, axis=1, keepdims=True)
            cand = jnp.where(pr == mx, iota, p_pairs)
            fi = jnp.min(cand, axis=1, keepdims=True)
            pr = jnp.where(iota == fi, -1.0, pr)
        msk = (pr < 0.0) & (p >= 0.0)
        mf = msk.astype(jnp.float32)
        pm = p * mf
        lg = jnp.log(1.0 + pm) * mf
        f0 = jnp.sum(pm, axis=1, keepdims=True)
        f1 = jnp.sum(bv * pm, axis=1, keepdims=True)
        f2 = jnp.sum(dv * pm, axis=1, keepdims=True)
        f3 = jnp.sum(bv * lg, axis=1, keepdims=True)
        f4 = jnp.sum(dv * lg, axis=1, keepdims=True)
        f5s = jnp.sum(jnp.exp(pm - 1.0) * (pm > 0).astype(jnp.float32) * mf,
                      axis=1, keepdims=True)
        f5 = jnp.log(jnp.exp(jnp.float32(-1.0)) + f5s) + 1.0
        feats = jnp.concatenate([f0, f1, f2, f3, f4, f5], axis=1)
        f3d = feats.reshape(3, num_steps * gb, 6)
        mu2 = jnp.mean(f3d, axis=1, keepdims=True)
        var2 = jnp.mean((f3d - mu2) ** 2, axis=1, keepdims=True)
        fn = (f3d - mu2) / jnp.sqrt(var2 + EPS)
        t = fn * pw_ref[...][:, None, :]
        s = jnp.sum(jnp.sum(t, axis=2, keepdims=True), axis=0)
        out_ref[:, :] = s + pb_ref[...]


def kernel(eigenvalues, eigenvectors_sq, W1, b1, W2, b2, W3, b3,
           proj_W, proj_b, birth_idx, death_idx):
    L, V, N = eigenvectors_sq.shape
    P = birth_idx.shape[-1]
    planes = W1.shape[1]
    groups = W2.shape[1]
    rows_total = L * 6 * P

    info = plsc.get_sparse_core_info()
    nc = info.num_cores
    nw = nc * info.num_subcores
    rpw = rows_total // nw
    nch = rpw // IDX_CHUNK

    all_idx = jnp.concatenate([birth_idx, death_idx], axis=1)
    goff = (jnp.arange(L, dtype=jnp.int32) * V)[:, None, None]
    idx_arr = (all_idx + goff).reshape(nw, nch, IDX_CHUNK)

    table = eigenvectors_sq.reshape(L * V, N)
    rows = _sc_gather_build(rows_total, rpw, nch, N, nc)(table, idx_arr)
    rows4 = rows.reshape(L, 6 * P, N)

    ev2 = eigenvalues[:, :, 0].reshape(L, 1, N)
    w1t = W1.T
    b1c = b1.reshape(planes, 1)
    w2t = W2.T
    b2c = b2.reshape(groups, 1)
    w3t = W3.T
    b3c = b3.reshape(1, 1)
    pw3 = proj_W.reshape(3, 6)
    pb2 = proj_b.reshape(1, 1)

    gb = 4
    return pl.pallas_call(
        _tc_body,
        grid=(L // gb,),
        in_specs=[
            pl.BlockSpec((gb, 1, N), lambda l: (l, 0, 0)),
            pl.BlockSpec((planes, 1), lambda l: (0, 0)),
            pl.BlockSpec((planes, 1), lambda l: (0, 0)),
            pl.BlockSpec((groups, planes), lambda l: (0, 0)),
            pl.BlockSpec((groups, 1), lambda l: (0, 0)),
            pl.BlockSpec((1, groups), lambda l: (0, 0)),
            pl.BlockSpec((1, 1), lambda l: (0, 0)),
            pl.BlockSpec((3, 6), lambda l: (0, 0)),
            pl.BlockSpec((1, 1), lambda l: (0, 0)),
            pl.BlockSpec((gb, 6 * P, N), lambda l: (l, 0, 0)),
        ],
        out_specs=pl.BlockSpec((L, 1), lambda l: (0, 0)),
        out_shape=jax.ShapeDtypeStruct((L, 1), jnp.float32),
        scratch_shapes=[pltpu.VMEM((L, 6 * P), jnp.float32)],
    )(ev2, w1t, b1c, w2t, b2c, w3t, b3c, pw3, pb2, rows4)


# speedup vs baseline: 1.0166x; 1.0098x over previous
import functools

import jax
import jax.numpy as jnp
from jax import lax
from jax.experimental import pallas as pl
from jax.experimental.pallas import tpu as pltpu
from jax.experimental.pallas import tpu_sc as plsc

TOPK = 20
EPS = 1e-5
IDX_CHUNK = 128


def _sc_gather_build(rows_total, rpw, nch, n, nc):
    mesh = plsc.VectorSubcoreMesh(core_axis_name="c", subcore_axis_name="s")

    @functools.partial(
        pl.kernel,
        mesh=mesh,
        out_type=jax.ShapeDtypeStruct((rows_total, n), jnp.float32),
        scratch_types=[
            pltpu.VMEM((nch, IDX_CHUNK), jnp.int32),
            pltpu.VMEM((rpw, n), jnp.float32),
        ]
        + [pltpu.SemaphoreType.DMA],
    )
    def sc_gather(table_hbm, idx_hbm, out_hbm, idx_v, rows_v, sem):
        wid = lax.axis_index("s") * nc + lax.axis_index("c")
        pltpu.sync_copy(idx_hbm.at[wid], idx_v)
        copies = [
            pltpu.async_copy(
                table_hbm.at[idx_v.at[j]],
                rows_v.at[pl.ds(j * IDX_CHUNK, IDX_CHUNK)],
                sem,
            )
            for j in range(nch)
        ]
        for cp in copies:
            cp.wait()
        pltpu.sync_copy(rows_v, out_hbm.at[pl.ds(wid * rpw, rpw)])

    return sc_gather


def _tc_body(ev_ref, w1_ref, b1_ref, w2_ref, b2_ref, w3_ref, b3_ref,
             pw_ref, pb_ref, r_ref, out_ref, v_ref):
    step = pl.program_id(0)
    num_steps = pl.num_programs(0)
    gb = r_ref.shape[0]
    p_pairs = r_ref.shape[1] // 6

    for gi in range(gb):
        ev = ev_ref[gi]
        h1 = jnp.maximum(w1_ref[...] * ev + b1_ref[...], 0.0)
        h2 = lax.dot_general(w2_ref[...], h1, (((1,), (0,)), ((), ())),
                             preferred_element_type=jnp.float32)
        h2 = jnp.maximum(h2 + b2_ref[...], 0.0)
        g0 = lax.dot_general(w3_ref[...], h2, (((1,), (0,)), ((), ())),
                             preferred_element_type=jnp.float32)
        g0 = g0 + b3_ref[...]
        mu = jnp.mean(g0, axis=1, keepdims=True)
        var = jnp.mean((g0 - mu) ** 2, axis=1, keepdims=True)
        g = (g0 - mu) / jnp.sqrt(var + EPS)
        v = lax.dot_general(g, r_ref[gi], (((1,), (1,)), ((), ())),
                            preferred_element_type=jnp.float32)
        v_ref[pl.ds(step * gb + gi, 1), :] = v

    @pl.when(step == num_steps - 1)
    def _finish():
        vall = v_ref[:, :]
        bv = jnp.concatenate(
            [vall[:, c * p_pairs:(c + 1) * p_pairs] for c in range(3)], axis=0)
        dv = jnp.concatenate(
            [vall[:, (3 + c) * p_pairs:(4 + c) * p_pairs] for c in range(3)],
            axis=0)
        rows = bv.shape[0]
        p = jnp.abs(dv - bv)
        iota = lax.broadcasted_iota(jnp.int32, (rows, p_pairs), 1)

        pr = p
        for _ in range(TOPK):
            mx = jnp.max(pr, axis=1, keepdims=True)
            cand = jnp.where(pr == mx, iota, p_pairs)
            fi = jnp.min(cand, axis=1, keepdims=True)
            pr = jnp.where(iota == fi, -1.0, pr)
        msk = (pr < 0.0) & (p >= 0.0)
        mf = msk.astype(jnp.float32)
        pm = p * mf
        lg = jnp.log(1.0 + pm) * mf
        f0 = jnp.sum(pm, axis=1, keepdims=True)
        f1 = jnp.sum(bv * pm, axis=1, keepdims=True)
        f2 = jnp.sum(dv * pm, axis=1, keepdims=True)
        f3 = jnp.sum(bv * lg, axis=1, keepdims=True)
        f4 = jnp.sum(dv * lg, axis=1, keepdims=True)
        f5s = jnp.sum(jnp.exp(pm - 1.0) * (pm > 0).astype(jnp.float32) * mf,
                      axis=1, keepdims=True)
        f5 = jnp.log(jnp.exp(jnp.float32(-1.0)) + f5s) + 1.0
        feats = jnp.concatenate([f0, f1, f2, f3, f4, f5], axis=1)
        f3d = feats.reshape(3, num_steps * gb, 6)
        mu2 = jnp.mean(f3d, axis=1, keepdims=True)
        var2 = jnp.mean((f3d - mu2) ** 2, axis=1, keepdims=True)
        fn = (f3d - mu2) / jnp.sqrt(var2 + EPS)
        t = fn * pw_ref[...][:, None, :]
        s = jnp.sum(jnp.sum(t, axis=2, keepdims=True), axis=0)
        out_ref[:, :] = s + pb_ref[...]


def kernel(eigenvalues, eigenvectors_sq, W1, b1, W2, b2, W3, b3,
           proj_W, proj_b, birth_idx, death_idx):
    L, V, N = eigenvectors_sq.shape
    P = birth_idx.shape[-1]
    planes = W1.shape[1]
    groups = W2.shape[1]
    rows_total = L * 6 * P

    info = plsc.get_sparse_core_info()
    nc = info.num_cores
    nw = nc * info.num_subcores
    rpw = rows_total // nw
    nch = rpw // IDX_CHUNK

    all_idx = jnp.concatenate([birth_idx, death_idx], axis=1)
    goff = (jnp.arange(L, dtype=jnp.int32) * V)[:, None, None]
    idx_arr = (all_idx + goff).reshape(nw, nch, IDX_CHUNK)

    table = eigenvectors_sq.reshape(L * V, N)
    rows = _sc_gather_build(rows_total, rpw, nch, N, nc)(table, idx_arr)
    rows4 = rows.reshape(L, 6 * P, N)

    ev2 = eigenvalues[:, :, 0].reshape(L, 1, N)
    w1t = W1.T
    b1c = b1.reshape(planes, 1)
    w2t = W2.T
    b2c = b2.reshape(groups, 1)
    w3t = W3.T
    b3c = b3.reshape(1, 1)
    pw3 = proj_W.reshape(3, 6)
    pb2 = proj_b.reshape(1, 1)

    gb = 4
    return pl.pallas_call(
        _tc_body,
        grid=(L // gb,),
        in_specs=[
            pl.BlockSpec((gb, 1, N), lambda l: (l, 0, 0)),
            pl.BlockSpec((planes, 1), lambda l: (0, 0)),
            pl.BlockSpec((planes, 1), lambda l: (0, 0)),
            pl.BlockSpec((groups, planes), lambda l: (0, 0)),
            pl.BlockSpec((groups, 1), lambda l: (0, 0)),
            pl.BlockSpec((1, groups), lambda l: (0, 0)),
            pl.BlockSpec((1, 1), lambda l: (0, 0)),
            pl.BlockSpec((3, 6), lambda l: (0, 0)),
            pl.BlockSpec((1, 1), lambda l: (0, 0)),
            pl.BlockSpec((gb, 6 * P, N), lambda l: (l, 0, 0)),
        ],
        out_specs=pl.BlockSpec((L, 1), lambda l: (0, 0)),
        out_shape=jax.ShapeDtypeStruct((L, 1), jnp.float32),
        scratch_shapes=[pltpu.VMEM((L, 6 * P), jnp.float32)],
    )(ev2, w1t, b1c, w2t, b2c, w3t, b3c, pw3, pb2, rows4)
